# Initial kernel scaffold; baseline (speedup 1.0000x reference)
import jax
import jax.numpy as jnp
from jax.experimental import pallas as pl


def kernel(feat, edge_index, in_norm, W1, b1, W2, b2, W_ih, W_hh, b_ih, b_hh):
    raise NotImplementedError("write your pallas kernel here")


# speedup vs baseline: 5.5003x; 5.5003x over previous
import functools

import jax
import jax.numpy as jnp
from jax import lax
from jax.experimental import pallas as pl
from jax.experimental.pallas import tpu as pltpu
from jax.experimental.pallas import tpu_sc as plsc

N = 10000
D = 128
KCAP = 2048
BLK = 512
CHUNK = 128
NW = 32


def _sc_gather_rows(table, idx, rows_per_worker):
  nchunks = rows_per_worker // CHUNK
  n_out = NW * rows_per_worker
  mesh = plsc.VectorSubcoreMesh(core_axis_name="c", subcore_axis_name="s")

  @functools.partial(
      pl.kernel,
      mesh=mesh,
      out_type=jax.ShapeDtypeStruct((n_out, D), jnp.float32),
      scratch_types=[
          pltpu.VMEM((CHUNK,), jnp.int32),
          pltpu.VMEM((CHUNK, D), jnp.float32),
          pltpu.SemaphoreType.DMA,
      ],
  )
  def k(table_hbm, idx_hbm, out_hbm, idx_v, rows_v, sem):
    wid = lax.axis_index("s") * 2 + lax.axis_index("c")
    base = wid * rows_per_worker

    def body(i, carry):
      off = base + i * CHUNK
      pltpu.sync_copy(idx_hbm.at[pl.ds(off, CHUNK)], idx_v)
      pltpu.async_copy(table_hbm.at[idx_v], rows_v, sem).wait()
      pltpu.sync_copy(rows_v, out_hbm.at[pl.ds(off, CHUNK)])
      return carry

    lax.fori_loop(0, nchunks, body, 0)

  return k(table, idx)


def _lstm_body(a_smem, msgs_hbm, wih_ref, whh_ref, bias_ref, invd_ref,
               out_ref, h_ref, c_ref, xs_ref, sem):
  h_ref[...] = jnp.zeros_like(h_ref)
  c_ref[...] = jnp.zeros_like(c_ref)

  def step_cond(carry):
    t, _ = carry
    return jnp.logical_and(t < KCAP, a_smem[t] > 0)

  def step_body(carry):
    t, row_off = carry
    a_t = a_smem[t]
    nblk = (a_t + (BLK - 1)) // BLK

    def blk_body(b, carry2):
      r0 = b * BLK
      cp = pltpu.make_async_copy(
          msgs_hbm.at[pl.ds(row_off + r0, BLK)], xs_ref, sem)
      cp.start()
      cp.wait()
      xs = xs_ref[...]
      hb = h_ref[pl.ds(r0, BLK), :]
      cb = c_ref[pl.ds(r0, BLK), :]
      gates = (
          jnp.dot(xs, wih_ref[...], preferred_element_type=jnp.float32)
          + jnp.dot(hb, whh_ref[...], preferred_element_type=jnp.float32)
          + bias_ref[...])
      ii = jax.nn.sigmoid(gates[:, 0:D])
      ff = jax.nn.sigmoid(gates[:, D:2 * D])
      gg = jnp.tanh(gates[:, 2 * D:3 * D])
      oo = jax.nn.sigmoid(gates[:, 3 * D:4 * D])
      cn = ff * cb + ii * gg
      hn = oo * jnp.tanh(cn)
      m = (r0 + lax.broadcasted_iota(jnp.int32, (BLK, 1), 0)) < a_t
      h_ref[pl.ds(r0, BLK), :] = jnp.where(m, hn, hb)
      c_ref[pl.ds(r0, BLK), :] = jnp.where(m, cn, cb)
      return carry2

    lax.fori_loop(0, nblk, blk_body, 0)
    return (t + 1, row_off + a_t)

  lax.while_loop(step_cond, step_body, (jnp.int32(0), jnp.int32(0)))
  out_ref[...] = h_ref[0:N, :] * invd_ref[...]


def _final_body(feat_ref, ah_ref, w1_ref, w2_ref, bias_ref, o_ref):
  o_ref[...] = (
      jnp.dot(feat_ref[...], w1_ref[...], preferred_element_type=jnp.float32)
      + jnp.dot(ah_ref[...], w2_ref[...], preferred_element_type=jnp.float32)
      + bias_ref[...])


def kernel(feat, edge_index, in_norm, W1, b1, W2, b2, W_ih, W_hh, b_ih, b_hh):
  del in_norm
  E = edge_index.shape[1]
  src = edge_index[0]
  dst = edge_index[1]

  counts = jnp.bincount(dst, length=N).astype(jnp.int32)
  node_perm = jnp.argsort(-counts)
  sorted_counts = counts[node_perm]
  rank = jnp.zeros((N,), jnp.int32).at[node_perm].set(
      jnp.arange(N, dtype=jnp.int32))

  order = jnp.argsort(dst)
  sdst = dst[order]
  ssrc = src[order]
  starts = jnp.concatenate(
      [jnp.zeros((1,), jnp.int32), jnp.cumsum(counts)[:-1].astype(jnp.int32)])
  pos = jnp.arange(E, dtype=jnp.int32) - starts[sdst]

  cd = jnp.bincount(jnp.minimum(counts, KCAP), length=KCAP + 1)
  a_steps = (N - jnp.cumsum(cd)[:KCAP]).astype(jnp.int32)
  offs = jnp.concatenate(
      [jnp.zeros((1,), jnp.int32), jnp.cumsum(a_steps)[:-1].astype(jnp.int32)])

  rows_per_worker = ((E + NW * CHUNK - 1) // (NW * CHUNK)) * CHUNK
  e_pad = NW * rows_per_worker
  sm_pos = offs[pos] + rank[sdst]
  gsrc = jnp.zeros((e_pad,), jnp.int32).at[sm_pos].set(ssrc)

  inv_deg_rank = (
      1.0 / jnp.maximum(sorted_counts, 1).astype(jnp.float32))[:, None]

  msgs = _sc_gather_rows(feat, gsrc, rows_per_worker)

  n_pad = ((N + BLK - 1) // BLK) * BLK
  wihT = W_ih.T
  whhT = W_hh.T
  bias = (b_ih + b_hh)[None, :]
  ah_rank = pl.pallas_call(
      _lstm_body,
      out_shape=jax.ShapeDtypeStruct((N, D), jnp.float32),
      in_specs=[
          pl.BlockSpec(memory_space=pltpu.SMEM),
          pl.BlockSpec(memory_space=pl.ANY),
          pl.BlockSpec(memory_space=pltpu.VMEM),
          pl.BlockSpec(memory_space=pltpu.VMEM),
          pl.BlockSpec(memory_space=pltpu.VMEM),
          pl.BlockSpec(memory_space=pltpu.VMEM),
      ],
      out_specs=pl.BlockSpec(memory_space=pltpu.VMEM),
      scratch_shapes=[
          pltpu.VMEM((n_pad, D), jnp.float32),
          pltpu.VMEM((n_pad, D), jnp.float32),
          pltpu.VMEM((BLK, D), jnp.float32),
          pltpu.SemaphoreType.DMA,
      ],
  )(a_steps, msgs, wihT, whhT, bias, inv_deg_rank)

  rpw_c = ((N + NW * CHUNK - 1) // (NW * CHUNK)) * CHUNK
  rank_pad = jnp.concatenate(
      [rank, jnp.zeros((NW * rpw_c - N,), jnp.int32)])
  ah_nat = _sc_gather_rows(ah_rank, rank_pad, rpw_c)[:N]

  fin_blk = 1000
  out = pl.pallas_call(
      _final_body,
      grid=(N // fin_blk,),
      out_shape=jax.ShapeDtypeStruct((N, D), jnp.float32),
      in_specs=[
          pl.BlockSpec((fin_blk, D), lambda i: (i, 0)),
          pl.BlockSpec((fin_blk, D), lambda i: (i, 0)),
          pl.BlockSpec((D, D), lambda i: (0, 0)),
          pl.BlockSpec((D, D), lambda i: (0, 0)),
          pl.BlockSpec((1, D), lambda i: (0, 0)),
      ],
      out_specs=pl.BlockSpec((fin_blk, D), lambda i: (i, 0)),
  )(feat, ah_nat, W1.T, W2.T, (b1 + b2)[None, :])
  return out


# speedup vs baseline: 14.5545x; 2.6461x over previous
import functools

import jax
import jax.numpy as jnp
from jax import lax
from jax.experimental import pallas as pl
from jax.experimental.pallas import tpu as pltpu
from jax.experimental.pallas import tpu_sc as plsc

N = 10000
D = 128
KCAP = 2048
BLK = 512
CHUNK = 128
NW = 32


def _sc_scatter_msgs(sdst, ssrc, starts, rank, offs, feat, e_real, e_pad):
  rows_per_worker = e_pad // NW
  nchunks = rows_per_worker // CHUNK
  mesh = plsc.VectorSubcoreMesh(core_axis_name="c", subcore_axis_name="s")

  @functools.partial(
      pl.kernel,
      mesh=mesh,
      out_type=jax.ShapeDtypeStruct((e_pad, D), jnp.float32),
      scratch_types=[
          pltpu.VMEM((CHUNK,), jnp.int32),
          pltpu.VMEM((CHUNK,), jnp.int32),
          pltpu.VMEM((CHUNK,), jnp.int32),
          pltpu.VMEM((CHUNK,), jnp.int32),
          pltpu.VMEM((CHUNK,), jnp.int32),
          pltpu.VMEM((CHUNK,), jnp.int32),
          pltpu.VMEM((CHUNK,), jnp.int32),
          pltpu.VMEM((CHUNK, D), jnp.float32),
          pltpu.SemaphoreType.DMA,
      ],
  )
  def k(sdst_hbm, ssrc_hbm, starts_hbm, rank_hbm, offs_hbm, feat_hbm, out_hbm,
        sdst_v, ssrc_v, st_v, rk_v, pos_v, of_v, smpos_v, rows_v, sem):
    wid = lax.axis_index("s") * 2 + lax.axis_index("c")
    base = wid * rows_per_worker

    def body(i, carry):
      off = base + i * CHUNK
      pltpu.sync_copy(sdst_hbm.at[pl.ds(off, CHUNK)], sdst_v)
      pltpu.sync_copy(ssrc_hbm.at[pl.ds(off, CHUNK)], ssrc_v)
      pltpu.async_copy(starts_hbm.at[sdst_v], st_v, sem).wait()
      pltpu.async_copy(rank_hbm.at[sdst_v], rk_v, sem).wait()
      for j in range(CHUNK // 16):
        gidx = lax.iota(jnp.int32, 16) + (off + j * 16)
        st16 = st_v[pl.ds(j * 16, 16)]
        pos_v[pl.ds(j * 16, 16)] = jnp.clip(gidx - st16, 0, KCAP - 1)
      pltpu.async_copy(offs_hbm.at[pos_v], of_v, sem).wait()
      for j in range(CHUNK // 16):
        gidx = lax.iota(jnp.int32, 16) + (off + j * 16)
        of16 = of_v[pl.ds(j * 16, 16)]
        rk16 = rk_v[pl.ds(j * 16, 16)]
        smpos_v[pl.ds(j * 16, 16)] = jnp.where(
            gidx >= e_real, gidx, of16 + rk16)
      pltpu.async_copy(feat_hbm.at[ssrc_v], rows_v, sem).wait()
      pltpu.async_copy(rows_v, out_hbm.at[smpos_v], sem).wait()
      return carry

    lax.fori_loop(0, nchunks, body, 0)

  return k(sdst, ssrc, starts, rank, offs, feat)


def _sc_gather_rows(table, idx, rows_per_worker):
  nchunks = rows_per_worker // CHUNK
  n_out = NW * rows_per_worker
  mesh = plsc.VectorSubcoreMesh(core_axis_name="c", subcore_axis_name="s")

  @functools.partial(
      pl.kernel,
      mesh=mesh,
      out_type=jax.ShapeDtypeStruct((n_out, D), jnp.float32),
      scratch_types=[
          pltpu.VMEM((CHUNK,), jnp.int32),
          pltpu.VMEM((CHUNK, D), jnp.float32),
          pltpu.SemaphoreType.DMA,
      ],
  )
  def k(table_hbm, idx_hbm, out_hbm, idx_v, rows_v, sem):
    wid = lax.axis_index("s") * 2 + lax.axis_index("c")
    base = wid * rows_per_worker

    def body(i, carry):
      off = base + i * CHUNK
      pltpu.sync_copy(idx_hbm.at[pl.ds(off, CHUNK)], idx_v)
      pltpu.async_copy(table_hbm.at[idx_v], rows_v, sem).wait()
      pltpu.sync_copy(rows_v, out_hbm.at[pl.ds(off, CHUNK)])
      return carry

    lax.fori_loop(0, nchunks, body, 0)

  return k(table, idx)


def _lstm_body(a_smem, msgs_hbm, wih_ref, whh_ref, bias_ref, invd_ref,
               out_ref, h_ref, c_ref, xs_ref, sem):
  h_ref[...] = jnp.zeros_like(h_ref)
  c_ref[...] = jnp.zeros_like(c_ref)

  def step_cond(carry):
    t, _ = carry
    return jnp.logical_and(t < KCAP, a_smem[t] > 0)

  def step_body(carry):
    t, row_off = carry
    a_t = a_smem[t]
    nblk = (a_t + (BLK - 1)) // BLK

    def blk_body(b, carry2):
      r0 = b * BLK
      cp = pltpu.make_async_copy(
          msgs_hbm.at[pl.ds(row_off + r0, BLK)], xs_ref, sem)
      cp.start()
      cp.wait()
      xs = xs_ref[...]
      hb = h_ref[pl.ds(r0, BLK), :]
      cb = c_ref[pl.ds(r0, BLK), :]
      gates = (
          jnp.dot(xs, wih_ref[...], preferred_element_type=jnp.float32)
          + jnp.dot(hb, whh_ref[...], preferred_element_type=jnp.float32)
          + bias_ref[...])
      ii = jax.nn.sigmoid(gates[:, 0:D])
      ff = jax.nn.sigmoid(gates[:, D:2 * D])
      gg = jnp.tanh(gates[:, 2 * D:3 * D])
      oo = jax.nn.sigmoid(gates[:, 3 * D:4 * D])
      cn = ff * cb + ii * gg
      hn = oo * jnp.tanh(cn)
      m = (r0 + lax.broadcasted_iota(jnp.int32, (BLK, 1), 0)) < a_t
      h_ref[pl.ds(r0, BLK), :] = jnp.where(m, hn, hb)
      c_ref[pl.ds(r0, BLK), :] = jnp.where(m, cn, cb)
      return carry2

    lax.fori_loop(0, nblk, blk_body, 0)
    return (t + 1, row_off + a_t)

  lax.while_loop(step_cond, step_body, (jnp.int32(0), jnp.int32(0)))
  out_ref[...] = h_ref[0:N, :] * invd_ref[...]


def _final_body(feat_ref, ah_ref, w1_ref, w2_ref, bias_ref, o_ref):
  o_ref[...] = (
      jnp.dot(feat_ref[...], w1_ref[...], preferred_element_type=jnp.float32)
      + jnp.dot(ah_ref[...], w2_ref[...], preferred_element_type=jnp.float32)
      + bias_ref[...])


def kernel(feat, edge_index, in_norm, W1, b1, W2, b2, W_ih, W_hh, b_ih, b_hh):
  del in_norm
  E = edge_index.shape[1]
  src = edge_index[0]
  dst = edge_index[1]

  counts = jnp.bincount(dst, length=N).astype(jnp.int32)
  neg_sorted, node_perm = lax.sort(
      (-counts, jnp.arange(N, dtype=jnp.int32)), num_keys=1, is_stable=True)
  sorted_counts = -neg_sorted
  rank = jnp.zeros((N,), jnp.int32).at[node_perm].set(
      jnp.arange(N, dtype=jnp.int32))
  sdst, ssrc = lax.sort((dst, src), num_keys=1, is_stable=True)
  starts = jnp.concatenate(
      [jnp.zeros((1,), jnp.int32), jnp.cumsum(counts)[:-1].astype(jnp.int32)])

  cd = jnp.bincount(jnp.minimum(counts, KCAP), length=KCAP + 1)
  a_steps = (N - jnp.cumsum(cd)[:KCAP]).astype(jnp.int32)
  offs = jnp.concatenate(
      [jnp.zeros((1,), jnp.int32), jnp.cumsum(a_steps)[:-1].astype(jnp.int32)])

  e_pad = ((E + NW * CHUNK - 1) // (NW * CHUNK)) * (NW * CHUNK)
  pad = e_pad - E
  sdst_p = jnp.concatenate([sdst, jnp.zeros((pad,), jnp.int32)])
  ssrc_p = jnp.concatenate([ssrc, jnp.zeros((pad,), jnp.int32)])

  inv_deg_rank = (
      1.0 / jnp.maximum(sorted_counts, 1).astype(jnp.float32))[:, None]

  msgs = _sc_scatter_msgs(sdst_p, ssrc_p, starts, rank, offs, feat, E, e_pad)

  n_pad = ((N + BLK - 1) // BLK) * BLK
  wihT = W_ih.T
  whhT = W_hh.T
  bias = (b_ih + b_hh)[None, :]
  ah_rank = pl.pallas_call(
      _lstm_body,
      out_shape=jax.ShapeDtypeStruct((N, D), jnp.float32),
      in_specs=[
          pl.BlockSpec(memory_space=pltpu.SMEM),
          pl.BlockSpec(memory_space=pl.ANY),
          pl.BlockSpec(memory_space=pltpu.VMEM),
          pl.BlockSpec(memory_space=pltpu.VMEM),
          pl.BlockSpec(memory_space=pltpu.VMEM),
          pl.BlockSpec(memory_space=pltpu.VMEM),
      ],
      out_specs=pl.BlockSpec(memory_space=pltpu.VMEM),
      scratch_shapes=[
          pltpu.VMEM((n_pad, D), jnp.float32),
          pltpu.VMEM((n_pad, D), jnp.float32),
          pltpu.VMEM((BLK, D), jnp.float32),
          pltpu.SemaphoreType.DMA,
      ],
  )(a_steps, msgs, wihT, whhT, bias, inv_deg_rank)

  rpw_c = ((N + NW * CHUNK - 1) // (NW * CHUNK)) * CHUNK
  rank_pad = jnp.concatenate(
      [rank, jnp.zeros((NW * rpw_c - N,), jnp.int32)])
  ah_nat = _sc_gather_rows(ah_rank, rank_pad, rpw_c)[:N]

  fin_blk = 1000
  out = pl.pallas_call(
      _final_body,
      grid=(N // fin_blk,),
      out_shape=jax.ShapeDtypeStruct((N, D), jnp.float32),
      in_specs=[
          pl.BlockSpec((fin_blk, D), lambda i: (i, 0)),
          pl.BlockSpec((fin_blk, D), lambda i: (i, 0)),
          pl.BlockSpec((D, D), lambda i: (0, 0)),
          pl.BlockSpec((D, D), lambda i: (0, 0)),
          pl.BlockSpec((1, D), lambda i: (0, 0)),
      ],
      out_specs=pl.BlockSpec((fin_blk, D), lambda i: (i, 0)),
  )(feat, ah_nat, W1.T, W2.T, (b1 + b2)[None, :])
  return out
